# baseline (speedup 1.0000x reference)
import functools

import jax
import jax.numpy as jnp
from jax.experimental import pallas as pl
from jax.experimental.pallas import tpu as pltpu

N = 10000
D = 128
RB = 1000
MB = 200


def _mm_small_kernel(x_ref, w_ref, o_ref):
    o_ref[...] = jnp.dot(x_ref[...], w_ref[...],
                         preferred_element_type=jnp.float32)


def _small_matmul(x, w):
    return pl.pallas_call(
        _mm_small_kernel,
        grid=(N // RB,),
        in_specs=[pl.BlockSpec((RB, D), lambda i: (i, 0)),
                  pl.BlockSpec((D, D), lambda i: (0, 0))],
        out_specs=pl.BlockSpec((RB, D), lambda i: (i, 0)),
        out_shape=jax.ShapeDtypeStruct((N, D), jnp.float32),
    )(x, w)


def _spmm_kernel(a_ref, s_ref, o_ref, *, relu):
    acc = jnp.dot(a_ref[...], s_ref[...], preferred_element_type=jnp.float32)
    if relu:
        acc = jnp.maximum(acc, 0.0)
    o_ref[...] = acc


def _big_matmul(a, s, relu):
    return pl.pallas_call(
        functools.partial(_spmm_kernel, relu=relu),
        grid=(N // MB,),
        in_specs=[pl.BlockSpec((MB, N), lambda i: (i, 0)),
                  pl.BlockSpec((N, D), lambda i: (0, 0))],
        out_specs=pl.BlockSpec((MB, D), lambda i: (i, 0)),
        out_shape=jax.ShapeDtypeStruct((N, D), jnp.float32),
        compiler_params=pltpu.CompilerParams(
            dimension_semantics=("arbitrary",)),
    )(a, s)


def kernel(x, adjacency, W0, W1):
    s0 = _small_matmul(x, W0)
    h = _big_matmul(adjacency, s0, relu=True)
    t = _small_matmul(h, W1)
    return _big_matmul(adjacency, t, relu=False)
